# baseline (speedup 1.0000x reference)
import functools

import jax
import jax.numpy as jnp
from jax import lax
from jax.experimental import pallas as pl
from jax.experimental.pallas import tpu as pltpu
from jax.experimental.pallas import tpu_sc as plsc

_B = 16384
_D = 64
_A = 18
_NC = 2
_NS = 16
_NW = _NC * _NS
_BPW = _B // _NW
_CH = 128
_NCH = _BPW // _CH
_L = 16


@functools.cache
def _make_gather_sc():
    @functools.partial(
        pl.kernel,
        out_type=jax.ShapeDtypeStruct((_B, _D), jnp.float32),
        mesh=plsc.VectorSubcoreMesh(core_axis_name="c", subcore_axis_name="s"),
        scratch_types=[
            pltpu.VMEM((3, _BPW), jnp.int32),
            pltpu.VMEM((_NCH, _CH), jnp.int32),
            pltpu.VMEM((_BPW, _D), jnp.float32),
            pltpu.SemaphoreType.DMA,
        ],
        compiler_params=pltpu.CompilerParams(use_tc_tiling_on_sc=False),
    )
    def _gather_sc(state_hbm, table_hbm, out_hbm, sv, idx_v, rows_v, sem):
        wid = lax.axis_index("s") * _NC + lax.axis_index("c")
        base = wid * _BPW
        pltpu.sync_copy(state_hbm.at[:, pl.ds(base, _BPW)], sv)
        for j in range(_NCH):
            for i in range(_CH // _L):
                sl = pl.ds(j * _CH + i * _L, _L)
                ids = sv[0, sl] * 10000 + sv[1, sl] * 100 + sv[2, sl]
                idx_v[j, pl.ds(i * _L, _L)] = ids
        copies = [
            pltpu.async_copy(
                table_hbm.at[idx_v.at[j]], rows_v.at[pl.ds(j * _CH, _CH)], sem
            )
            for j in range(_NCH)
        ]
        for c in copies:
            c.wait()
        pltpu.sync_copy(rows_v, out_hbm.at[pl.ds(base, _BPW)])

    return _gather_sc


_BM = 2048


def _head(e_ref, w_ref, b_ref, o_ref):
    o_ref[...] = (
        jnp.dot(e_ref[...], w_ref[...], preferred_element_type=jnp.float32)
        + b_ref[...]
    )


def kernel(state, embed_table, W, b):
    state_t = state.astype(jnp.int32).T
    e = _make_gather_sc()(state_t, embed_table)
    logits = pl.pallas_call(
        _head,
        grid=(_B // _BM,),
        in_specs=[
            pl.BlockSpec((_BM, _D), lambda i: (i, 0)),
            pl.BlockSpec((_D, _A), lambda i: (0, 0)),
            pl.BlockSpec((1, _A), lambda i: (0, 0)),
        ],
        out_specs=pl.BlockSpec((_BM, _A), lambda i: (i, 0)),
        out_shape=jax.ShapeDtypeStruct((_B, _A), jnp.float32),
    )(e, W, b.reshape(1, _A))
    return logits


# speedup vs baseline: 3.5457x; 3.5457x over previous
import functools

import jax
import jax.numpy as jnp
from jax import lax
from jax.experimental import pallas as pl
from jax.experimental.pallas import tpu as pltpu
from jax.experimental.pallas import tpu_sc as plsc

_B = 16384
_D = 64
_A = 18
_AP = 32
_N = 1_000_000
_NQ = 4
_NPACK = 1 << 18
_NC = 2
_NS = 16
_NW = _NC * _NS
_BPW = _B // _NW
_L = 16

_BN = 4096
_GRID = _NPACK // _BN
_LAST_BLK = _N // _BN


def _proj_body(t0_ref, t1_ref, t2_ref, t3_ref, w4_ref, b128_ref, o_ref):
    g = pl.program_id(0)
    start3 = jnp.minimum(3 * _GRID + g, _LAST_BLK) * _BN
    col = jax.lax.broadcasted_iota(jnp.int32, (1, _BN), 1)
    t3 = jnp.where(start3 + col < _N, t3_ref[...], 0.0)
    lhs = jnp.concatenate(
        [t0_ref[...], t1_ref[...], t2_ref[...], t3], axis=0
    )
    o_ref[...] = lax.dot_general(
        lhs, w4_ref[...], (((0,), (0,)), ((), ())),
        preferred_element_type=jnp.float32,
    ) + b128_ref[...]


@functools.cache
def _make_gather_sc():
    @functools.partial(
        pl.kernel,
        out_type=jax.ShapeDtypeStruct((_AP, _B), jnp.float32),
        mesh=plsc.VectorSubcoreMesh(core_axis_name="c", subcore_axis_name="s"),
        scratch_types=[
            pltpu.VMEM((3, _BPW), jnp.int32),
            pltpu.VMEM((4, 128), jnp.int32),
            pltpu.VMEM((4, 128), jnp.int32),
            pltpu.VMEM((_BPW, 128), jnp.float32),
            pltpu.VMEM((_AP, _BPW), jnp.float32),
            pltpu.SemaphoreType.DMA,
        ],
        compiler_params=pltpu.CompilerParams(needs_layout_passes=False),
    )
    def _k(state_hbm, tp_hbm, out_hbm, sv, idx_v, off_v, rows_v, dest_v, sem):
        wid = lax.axis_index("s") * _NC + lax.axis_index("c")
        base = wid * _BPW
        pltpu.sync_copy(state_hbm.at[:, pl.ds(base, _BPW)], sv)
        for j in range(4):
            for i in range(8):
                sl = pl.ds(j * 128 + i * _L, _L)
                ids = sv[0, sl] * 10000 + sv[1, sl] * 100 + sv[2, sl]
                idx_v[j, pl.ds(i * _L, _L)] = ids & (_NPACK - 1)
                off_v[j, pl.ds(i * _L, _L)] = (ids >> 18) * _AP
        copies = [
            pltpu.async_copy(
                tp_hbm.at[idx_v.at[j]], rows_v.at[pl.ds(j * 128, 128)], sem
            )
            for j in range(4)
        ]
        for c in copies:
            c.wait()
        lane = jax.lax.iota(jnp.int32, _L)
        for g in range(_BPW // _L):
            rows16 = lane + g * _L
            offs = off_v[g // 8, pl.ds((g % 8) * _L, _L)]
            for a in range(_AP):
                vals = plsc.load_gather(rows_v, [rows16, offs + a])
                dest_v[a, pl.ds(g * _L, _L)] = vals
        pltpu.sync_copy(dest_v, out_hbm.at[:, pl.ds(base, _BPW)])

    return _k


def kernel(state, embed_table, W, b):
    state_t = state.astype(jnp.int32).T
    table_t = embed_table.T
    w32 = jnp.zeros((_D, _AP), W.dtype).at[:, :_A].set(W)
    w4 = jnp.zeros((_NQ * _D, _NQ * _AP), W.dtype)
    for q in range(_NQ):
        w4 = w4.at[q * _D:(q + 1) * _D, q * _AP:(q + 1) * _AP].set(w32)
    b128 = jnp.tile(jnp.zeros((_AP,), b.dtype).at[:_A].set(b), _NQ)[None, :]
    tp = pl.pallas_call(
        _proj_body,
        grid=(_GRID,),
        in_specs=[
            pl.BlockSpec(
                (_D, _BN),
                lambda g, q=q: (0, jnp.minimum(q * _GRID + g, _LAST_BLK)),
            )
            for q in range(_NQ)
        ] + [
            pl.BlockSpec((_NQ * _D, 128), lambda g: (0, 0)),
            pl.BlockSpec((1, 128), lambda g: (0, 0)),
        ],
        out_specs=pl.BlockSpec((_BN, 128), lambda g: (g, 0)),
        out_shape=jax.ShapeDtypeStruct((_NPACK, 128), jnp.float32),
    )(table_t, table_t, table_t, table_t, w4, b128)
    out_t = _make_gather_sc()(state_t, tp)
    return out_t.T[:, :_A]


# speedup vs baseline: 3.7280x; 1.0514x over previous
import functools

import jax
import jax.numpy as jnp
from jax import lax
from jax.experimental import pallas as pl
from jax.experimental.pallas import tpu as pltpu
from jax.experimental.pallas import tpu_sc as plsc

_B = 16384
_D = 64
_A = 18
_AP = 32
_N = 1_000_000
_NQ = 4
_NPACK = 1 << 18
_NC = 2
_NS = 16
_NW = _NC * _NS
_BPW = _B // _NW
_L = 16

_BN = 8192
_GRID = _NPACK // _BN
_LAST_BLK = _N // _BN


def _proj_body(t0_ref, t1_ref, t2_ref, t3_ref, w4_ref, b128_ref, o_ref):
    g = pl.program_id(0)
    start3 = jnp.minimum(3 * _GRID + g, _LAST_BLK) * _BN
    col = jax.lax.broadcasted_iota(jnp.int32, (1, _BN), 1)
    t3 = jnp.where(start3 + col < _N, t3_ref[...], 0.0)
    lhs = jnp.concatenate(
        [t0_ref[...], t1_ref[...], t2_ref[...], t3], axis=0
    )
    o_ref[...] = lax.dot_general(
        lhs, w4_ref[...], (((0,), (0,)), ((), ())),
        preferred_element_type=jnp.float32,
    ) + b128_ref[...]


@functools.cache
def _make_gather_sc():
    @functools.partial(
        pl.kernel,
        out_type=jax.ShapeDtypeStruct((_AP, _B), jnp.float32),
        mesh=plsc.VectorSubcoreMesh(core_axis_name="c", subcore_axis_name="s"),
        scratch_types=[
            pltpu.VMEM((3, _BPW), jnp.int32),
            pltpu.VMEM((4, 128), jnp.int32),
            pltpu.VMEM((4, 128), jnp.int32),
            pltpu.VMEM((_BPW, 128), jnp.float32),
            pltpu.VMEM((_AP, _BPW), jnp.float32),
            pltpu.SemaphoreType.DMA,
        ],
        compiler_params=pltpu.CompilerParams(needs_layout_passes=False),
    )
    def _k(state_hbm, tp_hbm, out_hbm, sv, idx_v, off_v, rows_v, dest_v, sem):
        wid = lax.axis_index("s") * _NC + lax.axis_index("c")
        base = wid * _BPW
        pltpu.sync_copy(state_hbm.at[:, pl.ds(base, _BPW)], sv)
        for j in range(4):
            for i in range(8):
                sl = pl.ds(j * 128 + i * _L, _L)
                ids = sv[0, sl] * 10000 + sv[1, sl] * 100 + sv[2, sl]
                idx_v[j, pl.ds(i * _L, _L)] = ids & (_NPACK - 1)
                off_v[j, pl.ds(i * _L, _L)] = (ids >> 18) * _AP
        copies = [
            pltpu.async_copy(
                tp_hbm.at[idx_v.at[j]], rows_v.at[pl.ds(j * 128, 128)], sem
            )
            for j in range(4)
        ]
        for c in copies:
            c.wait()
        lane = jax.lax.iota(jnp.int32, _L)
        for g in range(_BPW // _L):
            rows16 = lane + g * _L
            offs = off_v[g // 8, pl.ds((g % 8) * _L, _L)]
            for a in range(_AP):
                vals = plsc.load_gather(rows_v, [rows16, offs + a])
                dest_v[a, pl.ds(g * _L, _L)] = vals
        pltpu.sync_copy(dest_v, out_hbm.at[:, pl.ds(base, _BPW)])

    return _k


def kernel(state, embed_table, W, b):
    state_t = state.astype(jnp.int32).T
    table_t = embed_table.T
    w32 = jnp.zeros((_D, _AP), W.dtype).at[:, :_A].set(W)
    w4 = jnp.zeros((_NQ * _D, _NQ * _AP), W.dtype)
    for q in range(_NQ):
        w4 = w4.at[q * _D:(q + 1) * _D, q * _AP:(q + 1) * _AP].set(w32)
    b128 = jnp.tile(jnp.zeros((_AP,), b.dtype).at[:_A].set(b), _NQ)[None, :]
    tp = pl.pallas_call(
        _proj_body,
        grid=(_GRID,),
        in_specs=[
            pl.BlockSpec(
                (_D, _BN),
                lambda g, q=q: (0, jnp.minimum(q * _GRID + g, _LAST_BLK)),
            )
            for q in range(_NQ)
        ] + [
            pl.BlockSpec((_NQ * _D, 128), lambda g: (0, 0)),
            pl.BlockSpec((1, 128), lambda g: (0, 0)),
        ],
        out_specs=pl.BlockSpec((_BN, 128), lambda g: (g, 0)),
        out_shape=jax.ShapeDtypeStruct((_NPACK, 128), jnp.float32),
    )(table_t, table_t, table_t, table_t, w4, b128)
    out_t = _make_gather_sc()(state_t, tp)
    return out_t.T[:, :_A]


# speedup vs baseline: 3.8221x; 1.0252x over previous
import functools

import jax
import jax.numpy as jnp
from jax import lax
from jax.experimental import pallas as pl
from jax.experimental.pallas import tpu as pltpu
from jax.experimental.pallas import tpu_sc as plsc

_B = 16384
_D = 64
_A = 18
_AP = 32
_N = 1_000_000
_NQ = 4
_NPACK = 1 << 18
_NC = 2
_NS = 16
_NW = _NC * _NS
_BPW = _B // _NW
_L = 16

_BN = 16384
_GRID = _NPACK // _BN
_LAST_BLK = _N // _BN


def _proj_body(t0_ref, t1_ref, t2_ref, t3_ref, w4_ref, b128_ref, o_ref):
    g = pl.program_id(0)
    start3 = jnp.minimum(3 * _GRID + g, _LAST_BLK) * _BN
    col = jax.lax.broadcasted_iota(jnp.int32, (1, _BN), 1)
    t3 = jnp.where(start3 + col < _N, t3_ref[...], 0.0)
    lhs = jnp.concatenate(
        [t0_ref[...], t1_ref[...], t2_ref[...], t3], axis=0
    )
    o_ref[...] = lax.dot_general(
        lhs, w4_ref[...], (((0,), (0,)), ((), ())),
        preferred_element_type=jnp.float32,
    ) + b128_ref[...]


@functools.cache
def _make_gather_sc():
    @functools.partial(
        pl.kernel,
        out_type=jax.ShapeDtypeStruct((_AP, _B), jnp.float32),
        mesh=plsc.VectorSubcoreMesh(core_axis_name="c", subcore_axis_name="s"),
        scratch_types=[
            pltpu.VMEM((3, _BPW), jnp.int32),
            pltpu.VMEM((4, 128), jnp.int32),
            pltpu.VMEM((4, 128), jnp.int32),
            pltpu.VMEM((_BPW, 128), jnp.float32),
            pltpu.VMEM((_AP, _BPW), jnp.float32),
            pltpu.SemaphoreType.DMA,
        ],
        compiler_params=pltpu.CompilerParams(needs_layout_passes=False),
    )
    def _k(state_hbm, tp_hbm, out_hbm, sv, idx_v, off_v, rows_v, dest_v, sem):
        wid = lax.axis_index("s") * _NC + lax.axis_index("c")
        base = wid * _BPW
        pltpu.sync_copy(state_hbm.at[:, pl.ds(base, _BPW)], sv)
        for j in range(4):
            for i in range(8):
                sl = pl.ds(j * 128 + i * _L, _L)
                ids = sv[0, sl] * 10000 + sv[1, sl] * 100 + sv[2, sl]
                idx_v[j, pl.ds(i * _L, _L)] = ids & (_NPACK - 1)
                off_v[j, pl.ds(i * _L, _L)] = (ids >> 18) * _AP
        copies = [
            pltpu.async_copy(
                tp_hbm.at[idx_v.at[j]], rows_v.at[pl.ds(j * 128, 128)], sem
            )
            for j in range(4)
        ]
        for c in copies:
            c.wait()
        lane = jax.lax.iota(jnp.int32, _L)
        for g in range(_BPW // _L):
            rows16 = lane + g * _L
            offs = off_v[g // 8, pl.ds((g % 8) * _L, _L)]
            for a in range(_AP):
                vals = plsc.load_gather(rows_v, [rows16, offs + a])
                dest_v[a, pl.ds(g * _L, _L)] = vals
        pltpu.sync_copy(dest_v, out_hbm.at[:, pl.ds(base, _BPW)])

    return _k


def kernel(state, embed_table, W, b):
    state_t = state.astype(jnp.int32).T
    table_t = embed_table.T
    w32 = jnp.zeros((_D, _AP), W.dtype).at[:, :_A].set(W)
    w4 = jnp.zeros((_NQ * _D, _NQ * _AP), W.dtype)
    for q in range(_NQ):
        w4 = w4.at[q * _D:(q + 1) * _D, q * _AP:(q + 1) * _AP].set(w32)
    b128 = jnp.tile(jnp.zeros((_AP,), b.dtype).at[:_A].set(b), _NQ)[None, :]
    tp = pl.pallas_call(
        _proj_body,
        grid=(_GRID,),
        in_specs=[
            pl.BlockSpec(
                (_D, _BN),
                lambda g, q=q: (0, jnp.minimum(q * _GRID + g, _LAST_BLK)),
            )
            for q in range(_NQ)
        ] + [
            pl.BlockSpec((_NQ * _D, 128), lambda g: (0, 0)),
            pl.BlockSpec((1, 128), lambda g: (0, 0)),
        ],
        out_specs=pl.BlockSpec((_BN, 128), lambda g: (g, 0)),
        out_shape=jax.ShapeDtypeStruct((_NPACK, 128), jnp.float32),
    )(table_t, table_t, table_t, table_t, w4, b128)
    out_t = _make_gather_sc()(state_t, tp)
    return out_t.T[:, :_A]


# speedup vs baseline: 4.5524x; 1.1911x over previous
import functools

import jax
import jax.numpy as jnp
from jax import lax
from jax.experimental import pallas as pl
from jax.experimental.pallas import tpu as pltpu
from jax.experimental.pallas import tpu_sc as plsc

_B = 16384
_D = 64
_A = 18
_N = 1_000_000
_NQ = 7
_NPACK = 147456
_NC = 2
_NS = 16
_NW = _NC * _NS
_BPW = _B // _NW
_L = 16

_BN = 8192
_GRID = _NPACK // _BN
_LAST_BLK = _N // _BN


def _proj_body(*refs):
    t_refs = refs[:_NQ]
    w7_ref, b128_ref, o_ref = refs[_NQ:]
    g = pl.program_id(0)
    col = jax.lax.broadcasted_iota(jnp.int32, (1, _BN), 1)
    slabs = []
    for q, t_ref in enumerate(t_refs):
        start = jnp.minimum(q * _GRID + g, _LAST_BLK) * _BN
        slabs.append(jnp.where(start + col < _N, t_ref[...], 0.0))
    lhs = jnp.concatenate(slabs, axis=0)
    o_ref[...] = lax.dot_general(
        lhs, w7_ref[...], (((0,), (0,)), ((), ())),
        preferred_element_type=jnp.float32,
    ) + b128_ref[...]


@functools.cache
def _make_gather_sc():
    @functools.partial(
        pl.kernel,
        out_type=jax.ShapeDtypeStruct((_A, _B), jnp.float32),
        mesh=plsc.VectorSubcoreMesh(core_axis_name="c", subcore_axis_name="s"),
        scratch_types=[
            pltpu.VMEM((3, _BPW), jnp.int32),
            pltpu.VMEM((4, 128), jnp.int32),
            pltpu.VMEM((4, 128), jnp.int32),
            pltpu.VMEM((_BPW, 128), jnp.float32),
            pltpu.VMEM((_A, _BPW), jnp.float32),
            pltpu.SemaphoreType.DMA,
        ],
        compiler_params=pltpu.CompilerParams(needs_layout_passes=False),
    )
    def _k(state_hbm, tp_hbm, out_hbm, sv, idx_v, off_v, rows_v, dest_v, sem):
        wid = lax.axis_index("s") * _NC + lax.axis_index("c")
        base = wid * _BPW
        pltpu.sync_copy(state_hbm.at[:, pl.ds(base, _BPW)], sv)
        for j in range(4):
            for i in range(8):
                sl = pl.ds(j * 128 + i * _L, _L)
                ids = sv[0, sl] * 10000 + sv[1, sl] * 100 + sv[2, sl]
                q = (ids >= _NPACK).astype(jnp.int32)
                for k in range(2, _NQ):
                    q = q + (ids >= k * _NPACK).astype(jnp.int32)
                idx_v[j, pl.ds(i * _L, _L)] = ids - q * _NPACK
                off_v[j, pl.ds(i * _L, _L)] = q * _A
        copies = [
            pltpu.async_copy(
                tp_hbm.at[idx_v.at[j]], rows_v.at[pl.ds(j * 128, 128)], sem
            )
            for j in range(4)
        ]
        for c in copies:
            c.wait()
        lane = jax.lax.iota(jnp.int32, _L)
        for g in range(_BPW // _L):
            rows16 = lane + g * _L
            offs = off_v[g // 8, pl.ds((g % 8) * _L, _L)]
            for a in range(_A):
                vals = plsc.load_gather(rows_v, [rows16, offs + a])
                dest_v[a, pl.ds(g * _L, _L)] = vals
        pltpu.sync_copy(dest_v, out_hbm.at[:, pl.ds(base, _BPW)])

    return _k


def kernel(state, embed_table, W, b):
    state_t = state.astype(jnp.int32).T
    table_t = embed_table.T
    w7 = jnp.zeros((_NQ * _D, 128), W.dtype)
    for q in range(_NQ):
        w7 = w7.at[q * _D:(q + 1) * _D, q * _A:(q + 1) * _A].set(W)
    b128 = jnp.concatenate([jnp.tile(b, _NQ), jnp.zeros((2,), b.dtype)])[None, :]
    tp = pl.pallas_call(
        _proj_body,
        grid=(_GRID,),
        in_specs=[
            pl.BlockSpec(
                (_D, _BN),
                lambda g, q=q: (0, jnp.minimum(q * _GRID + g, _LAST_BLK)),
            )
            for q in range(_NQ)
        ] + [
            pl.BlockSpec((_NQ * _D, 128), lambda g: (0, 0)),
            pl.BlockSpec((1, 128), lambda g: (0, 0)),
        ],
        out_specs=pl.BlockSpec((_BN, 128), lambda g: (g, 0)),
        out_shape=jax.ShapeDtypeStruct((_NPACK, 128), jnp.float32),
    )(*([table_t] * _NQ), w7, b128)
    out_t = _make_gather_sc()(state_t, tp)
    return out_t.T


# speedup vs baseline: 4.5847x; 1.0071x over previous
import functools

import jax
import jax.numpy as jnp
from jax import lax
from jax.experimental import pallas as pl
from jax.experimental.pallas import tpu as pltpu
from jax.experimental.pallas import tpu_sc as plsc

_B = 16384
_D = 64
_A = 18
_N = 1_000_000
_NQ = 7
_NPACK = 147456
_NC = 2
_NS = 16
_NW = _NC * _NS
_BPW = _B // _NW
_L = 16

_BN = 12288
_GRID = _NPACK // _BN
_LAST_BLK = _N // _BN


def _proj_body(*refs):
    t_refs = refs[:_NQ]
    w7_ref, b128_ref, o_ref = refs[_NQ:]
    g = pl.program_id(0)
    col = jax.lax.broadcasted_iota(jnp.int32, (1, _BN), 1)
    slabs = [t_ref[...] for t_ref in t_refs[:-1]]
    start = jnp.minimum((_NQ - 1) * _GRID + g, _LAST_BLK) * _BN
    slabs.append(jnp.where(start + col < _N, t_refs[-1][...], 0.0))
    lhs = jnp.concatenate(slabs, axis=0)
    o_ref[...] = lax.dot_general(
        lhs, w7_ref[...], (((0,), (0,)), ((), ())),
        preferred_element_type=jnp.float32,
    ) + b128_ref[...]


@functools.cache
def _make_gather_sc():
    @functools.partial(
        pl.kernel,
        out_type=jax.ShapeDtypeStruct((_A, _B), jnp.float32),
        mesh=plsc.VectorSubcoreMesh(core_axis_name="c", subcore_axis_name="s"),
        scratch_types=[
            pltpu.VMEM((3, _BPW), jnp.int32),
            pltpu.VMEM((4, 128), jnp.int32),
            pltpu.VMEM((4, 128), jnp.int32),
            pltpu.VMEM((_BPW, 128), jnp.float32),
            pltpu.VMEM((_A, _BPW), jnp.float32),
            pltpu.SemaphoreType.DMA,
        ],
        compiler_params=pltpu.CompilerParams(needs_layout_passes=False),
    )
    def _k(state_hbm, tp_hbm, out_hbm, sv, idx_v, off_v, rows_v, dest_v, sem):
        wid = lax.axis_index("s") * _NC + lax.axis_index("c")
        base = wid * _BPW
        pltpu.sync_copy(state_hbm.at[:, pl.ds(base, _BPW)], sv)
        for j in range(4):
            for i in range(8):
                sl = pl.ds(j * 128 + i * _L, _L)
                ids = sv[0, sl] * 10000 + sv[1, sl] * 100 + sv[2, sl]
                q = (ids >= _NPACK).astype(jnp.int32)
                for k in range(2, _NQ):
                    q = q + (ids >= k * _NPACK).astype(jnp.int32)
                idx_v[j, pl.ds(i * _L, _L)] = ids - q * _NPACK
                off_v[j, pl.ds(i * _L, _L)] = q * _A
        copies = [
            pltpu.async_copy(
                tp_hbm.at[idx_v.at[j]], rows_v.at[pl.ds(j * 128, 128)], sem
            )
            for j in range(4)
        ]
        for c in copies:
            c.wait()
        lane = jax.lax.iota(jnp.int32, _L)
        for g in range(_BPW // _L):
            rows16 = lane + g * _L
            offs = off_v[g // 8, pl.ds((g % 8) * _L, _L)]
            for a in range(_A):
                vals = plsc.load_gather(rows_v, [rows16, offs + a])
                dest_v[a, pl.ds(g * _L, _L)] = vals
        pltpu.sync_copy(dest_v, out_hbm.at[:, pl.ds(base, _BPW)])

    return _k


def kernel(state, embed_table, W, b):
    state_t = state.astype(jnp.int32).T
    table_t = embed_table.T
    w7 = jnp.zeros((_NQ * _D, 128), W.dtype)
    for q in range(_NQ):
        w7 = w7.at[q * _D:(q + 1) * _D, q * _A:(q + 1) * _A].set(W)
    b128 = jnp.concatenate([jnp.tile(b, _NQ), jnp.zeros((2,), b.dtype)])[None, :]
    tp = pl.pallas_call(
        _proj_body,
        grid=(_GRID,),
        in_specs=[
            pl.BlockSpec(
                (_D, _BN),
                (lambda g, q=q: (0, jnp.minimum(q * _GRID + g, _LAST_BLK)))
                if q == _NQ - 1 else (lambda g, q=q: (0, q * _GRID + g)),
            )
            for q in range(_NQ)
        ] + [
            pl.BlockSpec((_NQ * _D, 128), lambda g: (0, 0)),
            pl.BlockSpec((1, 128), lambda g: (0, 0)),
        ],
        out_specs=pl.BlockSpec((_BN, 128), lambda g: (g, 0)),
        out_shape=jax.ShapeDtypeStruct((_NPACK, 128), jnp.float32),
        compiler_params=pltpu.CompilerParams(
            vmem_limit_bytes=100 * 1024 * 1024
        ),
    )(*([table_t] * _NQ), w7, b128)
    out_t = _make_gather_sc()(state_t, tp)
    return out_t.T


# speedup vs baseline: 4.6545x; 1.0152x over previous
import functools

import jax
import jax.numpy as jnp
from jax import lax
from jax.experimental import pallas as pl
from jax.experimental.pallas import tpu as pltpu
from jax.experimental.pallas import tpu_sc as plsc

_B = 16384
_D = 64
_A = 18
_N = 1_000_000
_NQ = 7
_NPACK = 147456
_NC = 2
_NS = 16
_NW = _NC * _NS
_BPW = _B // _NW
_L = 16

_BN = 12288
_GRID = _NPACK // _BN
_LAST_BLK = _N // _BN


def _proj_body(*refs):
    t_refs = refs[:_NQ]
    w7_ref, b128_ref, o_ref = refs[_NQ:]
    g = pl.program_id(0)
    col = jax.lax.broadcasted_iota(jnp.int32, (1, _BN), 1)
    slabs = [t_ref[...] for t_ref in t_refs[:-1]]
    start = jnp.minimum((_NQ - 1) * _GRID + g, _LAST_BLK) * _BN
    slabs.append(jnp.where(start + col < _N, t_refs[-1][...], 0.0))
    lhs = jnp.concatenate(slabs, axis=0)
    o_ref[...] = lax.dot_general(
        lhs, w7_ref[...], (((0,), (0,)), ((), ())),
        preferred_element_type=jnp.float32,
    ) + b128_ref[...]


@functools.cache
def _make_gather_sc():
    @functools.partial(
        pl.kernel,
        out_type=jax.ShapeDtypeStruct((_A, _B), jnp.float32),
        mesh=plsc.VectorSubcoreMesh(core_axis_name="c", subcore_axis_name="s"),
        scratch_types=[
            pltpu.VMEM((3, _BPW), jnp.int32),
            pltpu.VMEM((4, 128), jnp.int32),
            pltpu.VMEM((4, 128), jnp.int32),
            pltpu.VMEM((_BPW, 128), jnp.float32),
            pltpu.VMEM((_A, _BPW), jnp.float32),
            pltpu.SemaphoreType.DMA,
            pltpu.SemaphoreType.DMA,
        ],
        compiler_params=pltpu.CompilerParams(needs_layout_passes=False),
    )
    def _k(state_hbm, tp_hbm, out_hbm, sv, idx_v, off_v, rows_v, dest_v, sem,
           osem):
        wid = lax.axis_index("s") * _NC + lax.axis_index("c")
        base = wid * _BPW
        pltpu.sync_copy(state_hbm.at[:, pl.ds(base, _BPW)], sv)
        copies = []
        for j in range(4):
            for i in range(8):
                sl = pl.ds(j * 128 + i * _L, _L)
                ids = sv[0, sl] * 10000 + sv[1, sl] * 100 + sv[2, sl]
                q = (ids >= _NPACK).astype(jnp.int32)
                for k in range(2, _NQ):
                    q = q + (ids >= k * _NPACK).astype(jnp.int32)
                idx_v[j, pl.ds(i * _L, _L)] = ids - q * _NPACK
                off_v[j, pl.ds(i * _L, _L)] = q * _A
            copies.append(pltpu.async_copy(
                tp_hbm.at[idx_v.at[j]], rows_v.at[pl.ds(j * 128, 128)], sem
            ))
        lane = jax.lax.iota(jnp.int32, _L)
        for j in range(4):
            copies[j].wait()
            for gi in range(8):
                g = j * 8 + gi
                rows16 = lane + g * _L
                offs = off_v[j, pl.ds(gi * _L, _L)]
                for a in range(_A):
                    vals = plsc.load_gather(rows_v, [rows16, offs + a])
                    dest_v[a, pl.ds(g * _L, _L)] = vals
            pltpu.async_copy(
                dest_v.at[:, pl.ds(j * 128, 128)],
                out_hbm.at[:, pl.ds(base + j * 128, 128)],
                osem,
            )
        pltpu.make_async_copy(
            out_hbm.at[:, pl.ds(0, _BPW)], dest_v, osem
        ).wait()

    return _k


def kernel(state, embed_table, W, b):
    state_t = state.astype(jnp.int32).T
    table_t = embed_table.T
    w7 = jnp.zeros((_NQ * _D, 128), W.dtype)
    for q in range(_NQ):
        w7 = w7.at[q * _D:(q + 1) * _D, q * _A:(q + 1) * _A].set(W)
    b128 = jnp.concatenate([jnp.tile(b, _NQ), jnp.zeros((2,), b.dtype)])[None, :]
    tp = pl.pallas_call(
        _proj_body,
        grid=(_GRID,),
        in_specs=[
            pl.BlockSpec(
                (_D, _BN),
                (lambda g, q=q: (0, jnp.minimum(q * _GRID + g, _LAST_BLK)))
                if q == _NQ - 1 else (lambda g, q=q: (0, q * _GRID + g)),
            )
            for q in range(_NQ)
        ] + [
            pl.BlockSpec((_NQ * _D, 128), lambda g: (0, 0)),
            pl.BlockSpec((1, 128), lambda g: (0, 0)),
        ],
        out_specs=pl.BlockSpec((_BN, 128), lambda g: (g, 0)),
        out_shape=jax.ShapeDtypeStruct((_NPACK, 128), jnp.float32),
        compiler_params=pltpu.CompilerParams(
            vmem_limit_bytes=100 * 1024 * 1024
        ),
    )(*([table_t] * _NQ), w7, b128)
    out_t = _make_gather_sc()(state_t, tp)
    return out_t.T
